# baseline (speedup 1.0000x reference)
import functools

import jax
import jax.numpy as jnp
from jax import lax
from jax.experimental import pallas as pl
from jax.experimental.pallas import tpu as pltpu
from jax.experimental.pallas import tpu_sc as plsc

C = 128
P = 40000
NX = 512
NCELL = 512 * 512
NTILES = 32
RANGE = NCELL // NTILES
CHUNK = 4000
NCHUNK = P // CHUNK
VPC = CHUNK // 16
CW = CHUNK * 4
DCH = 128
PAD_BASE = NCELL
CANVAS_ROWS = NCELL + NTILES * DCH
TCB = 4096


def _sc_body(feat_hbm, z_hbm, y_hbm, x_hbm, canvas_hbm, aux_hbm,
             cb0z, cb0y, cb0x, cb1z, cb1y, cb1x,
             aux, survp, survc2, rows0, rows1, rows2, rows3,
             sem_c0, sem_c1, sem_g0, sem_g1, sem_g2, sem_g3,
             sem_s0, sem_s1, sem_s2, sem_s3):
    cb0 = (cb0z, cb0y, cb0x)
    cb1 = (cb1z, cb1y, cb1x)
    wid = lax.axis_index("s") * 2 + lax.axis_index("c")
    base = wid * RANGE
    lanes = lax.iota(jnp.int32, 16)
    neg1 = jnp.full((16,), -1, jnp.int32)
    zero16 = jnp.zeros((16,), jnp.int32)

    def stage(ck, buf, sem):
        off = ck * CHUNK
        return [
            pltpu.make_async_copy(
                z_hbm.at[pl.ds(off, CHUNK)], buf[0], sem),
            pltpu.make_async_copy(
                y_hbm.at[pl.ds(off, CHUNK)], buf[1], sem),
            pltpu.make_async_copy(
                x_hbm.at[pl.ds(off, CHUNK)], buf[2], sem),
        ]

    for cp in stage(0, cb0, sem_c0):
        cp.start()

    @pl.loop(0, RANGE // 16, unroll=8)
    def _init(i):
        aux[pl.ds(i * 16, 16)] = neg1

    padc = PAD_BASE + wid * DCH
    @pl.loop(0, RANGE // 16 + 1, unroll=8)
    def _pad(i):
        survp[pl.ds(i * 16, 16)] = zero16
        survc2[(i * 16) // DCH, pl.ds((i * 16) % DCH, 16)] = (
            padc + (i % (DCH // 16)) * 16 + lanes)

    G = 5

    def scan_chunk(ck, buf):
        off = ck * CHUNK

        @pl.loop(0, VPC // G)
        def _vec(ii):
            i0 = ii * G
            zs = [buf[0][pl.ds((i0 + k) * 16, 16)] for k in range(G)]
            ys = [buf[1][pl.ds((i0 + k) * 16, 16)] for k in range(G)]
            xs = [buf[2][pl.ds((i0 + k) * 16, 16)] for k in range(G)]
            locs = [zs[k] + ys[k] * NX + xs[k] - base for k in range(G)]
            ms = [(l >= 0) & (l < RANGE) for l in locs]
            pvs = [off + (i0 + k) * 16 + lanes for k in range(G)]
            for k in range(G):
                plsc.store_scatter(aux, [locs[k]], pvs[k], mask=ms[k])
            for _pass in range(2):
                gots = [plsc.load_gather(aux, [locs[k]], mask=ms[k])
                        for k in range(G)]
                for k in range(G):
                    fix = ms[k] & (pvs[k] > gots[k])
                    plsc.store_scatter(aux, [locs[k]], pvs[k], mask=fix)

    with jax.named_scope("scan"):
        @pl.loop(0, NCHUNK, step=2)
        def _chunks(ck):
            for cp in stage(ck, cb0, sem_c0):
                cp.wait()

            @pl.when(ck + 1 < NCHUNK)
            def _():
                for cp in stage(ck + 1, cb1, sem_c1):
                    cp.start()
            scan_chunk(ck, cb0)
            for cp in stage(ck + 1, cb1, sem_c1):
                cp.wait()

            @pl.when(ck + 2 < NCHUNK)
            def _():
                for cp in stage(ck + 2, cb0, sem_c0):
                    cp.start()
            scan_chunk(ck + 1, cb1)

    def comp_body(i, cnt):
        v = aux[pl.ds(i * 16, 16)]
        m = v >= 0
        mi = m.astype(jnp.int32)
        cell = base + i * 16 + lanes
        tpos = cnt + plsc.cumsum(mi) - 1
        plsc.store_scatter(survp, [tpos], v, mask=m)
        plsc.store_scatter(
            survc2, [tpos >> 7, tpos & (DCH - 1)], cell, mask=m)
        return cnt + jnp.sum(mi)
    with jax.named_scope("compact"):
        cnt = lax.fori_loop(0, RANGE // 16, comp_body, 0, unroll=2)

    aux_pub = pltpu.make_async_copy(aux, aux_hbm.at[pl.ds(base, RANGE)], sem_c0)
    aux_pub.start()

    nch = (cnt + (DCH - 1)) // DCH
    bufs = (rows0, rows1, rows2, rows3)
    gsems = (sem_g0, sem_g1, sem_g2, sem_g3)
    ssems = (sem_s0, sem_s1, sem_s2, sem_s3)

    def g_copy(j, b):
        return pltpu.make_async_copy(
            feat_hbm.at[survp.at[pl.ds(j * DCH, DCH)]], bufs[b], gsems[b])

    def s_copy(j, b):
        return pltpu.make_async_copy(
            bufs[b], canvas_hbm.at[survc2.at[j]], ssems[b])

    @pl.when(nch > 0)
    def _():
        g_copy(0, 0).start()

    @pl.when(nch > 1)
    def _():
        g_copy(1, 1).start()

    def dma_body(j, _):
        for b in range(4):
            @pl.when(j % 4 == b)
            def _(b=b):
                g_copy(j, b).wait()

                @pl.when(j >= 2)
                def _():
                    s_copy(j - 2, (b + 2) % 4).wait()

                @pl.when(j + 2 < nch)
                def _():
                    g_copy(j + 2, (b + 2) % 4).start()
                s_copy(j, b).start()
        return 0
    with jax.named_scope("rowdma"):
        lax.fori_loop(0, nch, dma_body, 0)

    for b in range(4):
        @pl.when((nch > 1) & (lax.rem(nch - 2, 4) == b))
        def _(b=b):
            s_copy(nch - 2, b).wait()

        @pl.when((nch > 0) & (lax.rem(nch - 1, 4) == b))
        def _(b=b):
            s_copy(nch - 1, b).wait()

    aux_pub.wait()


def _tc_body(canvas_ref, aux_ref, out_ref):
    for r in range(TCB // NX):
        x = canvas_ref[pl.ds(r * NX, NX), :]
        m = (aux_ref[0, 0, pl.ds(r * NX, NX)] >= 0).reshape(1, NX)
        out_ref[:, r, :] = jnp.where(m, x.T, 0.0)


def kernel(pillar_features, voxel_coords):
    zcol = voxel_coords[:, 1]
    ycol = voxel_coords[:, 2]
    xcol = voxel_coords[:, 3]

    mesh = plsc.VectorSubcoreMesh(core_axis_name="c", subcore_axis_name="s")
    sc = functools.partial(
        pl.kernel,
        mesh=mesh,
        compiler_params=pltpu.CompilerParams(needs_layout_passes=False),
        out_type=(
            jax.ShapeDtypeStruct((CANVAS_ROWS, C), jnp.float32),
            jax.ShapeDtypeStruct((NCELL,), jnp.int32),
        ),
        scratch_types=[
            pltpu.VMEM((CHUNK,), jnp.int32),
            pltpu.VMEM((CHUNK,), jnp.int32),
            pltpu.VMEM((CHUNK,), jnp.int32),
            pltpu.VMEM((CHUNK,), jnp.int32),
            pltpu.VMEM((CHUNK,), jnp.int32),
            pltpu.VMEM((CHUNK,), jnp.int32),
            pltpu.VMEM((RANGE,), jnp.int32),
            pltpu.VMEM((RANGE + 16,), jnp.int32),
            pltpu.VMEM((RANGE // DCH + 1, DCH), jnp.int32),
            pltpu.VMEM((DCH, C), jnp.float32),
            pltpu.VMEM((DCH, C), jnp.float32),
            pltpu.VMEM((DCH, C), jnp.float32),
            pltpu.VMEM((DCH, C), jnp.float32),
        ] + [pltpu.SemaphoreType.DMA] * 10,
    )(_sc_body)
    canvas, aux = sc(pillar_features, zcol, ycol, xcol)

    out = pl.pallas_call(
        _tc_body,
        grid=(NCELL // TCB,),
        in_specs=[
            pl.BlockSpec((TCB, C), lambda b: (b, 0)),
            pl.BlockSpec((1, 1, TCB), lambda b: (b, 0, 0)),
        ],
        out_specs=pl.BlockSpec((C, TCB // NX, NX), lambda b: (0, b, 0)),
        out_shape=jax.ShapeDtypeStruct((C, NX, NX), jnp.float32),
    )(canvas, aux.reshape(NCELL // TCB, 1, TCB))
    return out
